# Initial kernel scaffold; baseline (speedup 1.0000x reference)
import jax
import jax.numpy as jnp
from jax.experimental import pallas as pl


def kernel(x, W):
    raise NotImplementedError("write your pallas kernel here")


# speedup vs baseline: 1.4243x; 1.4243x over previous
import functools

import jax
import jax.numpy as jnp
from jax import lax
from jax.experimental import pallas as pl
from jax.experimental.pallas import tpu as pltpu
from jax.experimental.pallas import tpu_sc as plsc

_COMMIT = 0.25
_K = 8192
_D = 32
_N = 16384
_BN = 256
_EPS = 1e-12
_IDX_CHUNK = 128


def _tc_body(x_ref, w_ref, idx_ref, wn_ref, loss_ref, wnt_ref, sw2_ref):
    step = pl.program_id(0)

    @pl.when(step == 0)
    def _init():
        w = w_ref[...]
        norm = jnp.sqrt(jnp.sum(w * w, axis=1, keepdims=True))
        wn = w / jnp.maximum(norm, _EPS)
        wn_ref[...] = wn
        wnt = wn.T
        sw2_ref[...] = jnp.sum(wnt * wnt, axis=0, keepdims=True)
        wnt_ref[...] = wnt.astype(jnp.bfloat16)
        loss_ref[...] = jnp.zeros_like(loss_ref)

    x = x_ref[...]
    xnorm = jnp.sqrt(jnp.sum(x * x, axis=1, keepdims=True))
    xn = x / jnp.maximum(xnorm, _EPS)
    sx2 = jnp.sum(xn * xn, axis=1, keepdims=True)
    scores = lax.dot_general(
        xn.astype(jnp.bfloat16), wnt_ref[...], (((1,), (0,)), ((), ())),
        preferred_element_type=jnp.float32)
    dist = (sx2 + sw2_ref[...]) - 2.0 * scores
    min_val = jnp.min(dist, axis=1, keepdims=True)
    col = lax.broadcasted_iota(jnp.int32, dist.shape, 1)
    cand = jnp.where(dist == min_val, col, jnp.int32(2 ** 30))
    idx_ref[...] = jnp.min(cand, axis=1, keepdims=True)
    loss_ref[...] += jnp.sum(min_val, axis=0, keepdims=True)


def _tc_call(x, W):
    grid = _N // _BN
    return pl.pallas_call(
        _tc_body,
        grid=(grid,),
        in_specs=[
            pl.BlockSpec((_BN, _D), lambda i: (i, 0)),
            pl.BlockSpec((_K, _D), lambda i: (0, 0)),
        ],
        out_specs=[
            pl.BlockSpec((_BN, 1), lambda i: (i, 0)),
            pl.BlockSpec((_K, _D), lambda i: (0, 0)),
            pl.BlockSpec((1, 1), lambda i: (0, 0)),
        ],
        out_shape=[
            jax.ShapeDtypeStruct((_N, 1), jnp.int32),
            jax.ShapeDtypeStruct((_K, _D), jnp.float32),
            jax.ShapeDtypeStruct((1, 1), jnp.float32),
        ],
        scratch_shapes=[
            pltpu.VMEM((_D, _K), jnp.bfloat16),
            pltpu.VMEM((1, _K), jnp.float32),
        ],
    )(x, W)


def _sc_gather(wn, idx):
    info = plsc.get_sparse_core_info()
    nw = info.num_cores * info.num_subcores
    bpw = _N // nw
    nchunk = bpw // _IDX_CHUNK
    mesh = plsc.VectorSubcoreMesh(core_axis_name="c", subcore_axis_name="s")

    @functools.partial(
        pl.kernel,
        mesh=mesh,
        out_type=jax.ShapeDtypeStruct((_N, _D), jnp.float32),
        scratch_types=[
            pltpu.VMEM((nchunk, _IDX_CHUNK), jnp.int32),
            pltpu.VMEM((nchunk, _IDX_CHUNK, _D), jnp.float32),
            pltpu.SemaphoreType.DMA,
        ],
        compiler_params=pltpu.CompilerParams(use_tc_tiling_on_sc=False),
    )
    def gather_kernel(wn_hbm, idx_hbm, out_hbm, idx_v, rows_v, sem):
        wid = lax.axis_index("s") * info.num_cores + lax.axis_index("c")
        base = wid * bpw
        for j in range(nchunk):
            pltpu.sync_copy(
                idx_hbm.at[pl.ds(base + j * _IDX_CHUNK, _IDX_CHUNK)],
                idx_v.at[j])
            pltpu.async_copy(wn_hbm.at[idx_v.at[j]], rows_v.at[j], sem).wait()
            pltpu.sync_copy(
                rows_v.at[j],
                out_hbm.at[pl.ds(base + j * _IDX_CHUNK, _IDX_CHUNK)])

    return gather_kernel(wn, idx)


def kernel(x, W):
    idx2d, wn, loss_sum = _tc_call(x, W)
    idx = idx2d[:, 0]
    quantized = _sc_gather(wn, idx)
    m = loss_sum[0, 0] / jnp.float32(_N * _D)
    loss = m + jnp.float32(_COMMIT) * m
    return (quantized, loss, idx)


# speedup vs baseline: 1.5481x; 1.0869x over previous
import functools

import jax
import jax.numpy as jnp
from jax import lax
from jax.experimental import pallas as pl
from jax.experimental.pallas import tpu as pltpu
from jax.experimental.pallas import tpu_sc as plsc

_COMMIT = 0.25
_K = 8192
_D = 32
_N = 16384
_BN = 512
_EPS = 1e-12
_IDX_CHUNK = 128


def _tc_body(x_ref, w_ref, idx_ref, wn_ref, loss_ref, wnt_ref, sw2_ref):
    step = pl.program_id(0)

    @pl.when(step == 0)
    def _init():
        w = w_ref[...]
        norm = jnp.sqrt(jnp.sum(w * w, axis=1, keepdims=True))
        wn = w / jnp.maximum(norm, _EPS)
        wn_ref[...] = wn
        wnt = wn.T
        sw2_ref[...] = jnp.sum(wnt * wnt, axis=0, keepdims=True)
        wnt_ref[...] = wnt.astype(jnp.bfloat16)
        loss_ref[...] = jnp.zeros_like(loss_ref)

    x = x_ref[...]
    xnorm = jnp.sqrt(jnp.sum(x * x, axis=1, keepdims=True))
    xn = x / jnp.maximum(xnorm, _EPS)
    sx2 = jnp.sum(xn * xn, axis=1, keepdims=True)
    scores = lax.dot_general(
        xn.astype(jnp.bfloat16), wnt_ref[...], (((1,), (0,)), ((), ())),
        preferred_element_type=jnp.float32)
    dist = (sx2 + sw2_ref[...]) - 2.0 * scores
    min_val = jnp.min(dist, axis=1, keepdims=True)
    col = lax.broadcasted_iota(jnp.int32, dist.shape, 1)
    cand = jnp.where(dist == min_val, col, jnp.int32(2 ** 30))
    idx_ref[...] = jnp.min(cand, axis=1, keepdims=True)
    loss_ref[...] += jnp.sum(min_val, axis=0, keepdims=True)


def _tc_call(x, W):
    grid = _N // _BN
    return pl.pallas_call(
        _tc_body,
        grid=(grid,),
        in_specs=[
            pl.BlockSpec((_BN, _D), lambda i: (i, 0)),
            pl.BlockSpec((_K, _D), lambda i: (0, 0)),
        ],
        out_specs=[
            pl.BlockSpec((_BN, 1), lambda i: (i, 0)),
            pl.BlockSpec((_K, _D), lambda i: (0, 0)),
            pl.BlockSpec((1, 1), lambda i: (0, 0)),
        ],
        out_shape=[
            jax.ShapeDtypeStruct((_N, 1), jnp.int32),
            jax.ShapeDtypeStruct((_K, _D), jnp.float32),
            jax.ShapeDtypeStruct((1, 1), jnp.float32),
        ],
        scratch_shapes=[
            pltpu.VMEM((_D, _K), jnp.bfloat16),
            pltpu.VMEM((1, _K), jnp.float32),
        ],
    )(x, W)


def _sc_gather(wn, idx):
    info = plsc.get_sparse_core_info()
    nw = info.num_cores * info.num_subcores
    bpw = _N // nw
    nchunk = bpw // _IDX_CHUNK
    mesh = plsc.VectorSubcoreMesh(core_axis_name="c", subcore_axis_name="s")

    @functools.partial(
        pl.kernel,
        mesh=mesh,
        out_type=jax.ShapeDtypeStruct((_N, _D), jnp.float32),
        scratch_types=[
            pltpu.VMEM((nchunk, _IDX_CHUNK), jnp.int32),
            pltpu.VMEM((nchunk, _IDX_CHUNK, _D), jnp.float32),
            pltpu.SemaphoreType.DMA,
        ],
        compiler_params=pltpu.CompilerParams(use_tc_tiling_on_sc=False),
    )
    def gather_kernel(wn_hbm, idx_hbm, out_hbm, idx_v, rows_v, sem):
        wid = lax.axis_index("s") * info.num_cores + lax.axis_index("c")
        base = wid * bpw
        for j in range(nchunk):
            pltpu.sync_copy(
                idx_hbm.at[pl.ds(base + j * _IDX_CHUNK, _IDX_CHUNK)],
                idx_v.at[j])
            pltpu.async_copy(wn_hbm.at[idx_v.at[j]], rows_v.at[j], sem).wait()
            pltpu.sync_copy(
                rows_v.at[j],
                out_hbm.at[pl.ds(base + j * _IDX_CHUNK, _IDX_CHUNK)])

    return gather_kernel(wn, idx)


def kernel(x, W):
    idx2d, wn, loss_sum = _tc_call(x, W)
    idx = idx2d[:, 0]
    quantized = _sc_gather(wn, idx)
    m = loss_sum[0, 0] / jnp.float32(_N * _D)
    loss = m + jnp.float32(_COMMIT) * m
    return (quantized, loss, idx)


# speedup vs baseline: 1.5606x; 1.0081x over previous
import functools

import jax
import jax.numpy as jnp
from jax import lax
from jax.experimental import pallas as pl
from jax.experimental.pallas import tpu as pltpu
from jax.experimental.pallas import tpu_sc as plsc

_COMMIT = 0.25
_K = 8192
_D = 32
_N = 16384
_BN = 1024
_EPS = 1e-12
_IDX_CHUNK = 128


def _tc_body(x_ref, w_ref, idx_ref, wn_ref, loss_ref, wnt_ref, sw2_ref):
    step = pl.program_id(0)

    @pl.when(step == 0)
    def _init():
        w = w_ref[...]
        norm = jnp.sqrt(jnp.sum(w * w, axis=1, keepdims=True))
        wn = w / jnp.maximum(norm, _EPS)
        wn_ref[...] = wn
        wnt = wn.T
        sw2_ref[...] = jnp.sum(wnt * wnt, axis=0, keepdims=True)
        wnt_ref[...] = wnt.astype(jnp.bfloat16)
        loss_ref[...] = jnp.zeros_like(loss_ref)

    x = x_ref[...]
    xnorm = jnp.sqrt(jnp.sum(x * x, axis=1, keepdims=True))
    xn = x / jnp.maximum(xnorm, _EPS)
    sx2 = jnp.sum(xn * xn, axis=1, keepdims=True)
    scores = lax.dot_general(
        xn.astype(jnp.bfloat16), wnt_ref[...], (((1,), (0,)), ((), ())),
        preferred_element_type=jnp.float32)
    dist = (sx2 + sw2_ref[...]) - 2.0 * scores
    min_val = jnp.min(dist, axis=1, keepdims=True)
    col = lax.broadcasted_iota(jnp.int32, dist.shape, 1)
    cand = jnp.where(dist == min_val, col, jnp.int32(2 ** 30))
    idx_ref[...] = jnp.min(cand, axis=1, keepdims=True)
    loss_ref[...] += jnp.sum(min_val, axis=0, keepdims=True)


def _tc_call(x, W):
    grid = _N // _BN
    return pl.pallas_call(
        _tc_body,
        grid=(grid,),
        in_specs=[
            pl.BlockSpec((_BN, _D), lambda i: (i, 0)),
            pl.BlockSpec((_K, _D), lambda i: (0, 0)),
        ],
        out_specs=[
            pl.BlockSpec((_BN, 1), lambda i: (i, 0)),
            pl.BlockSpec((_K, _D), lambda i: (0, 0)),
            pl.BlockSpec((1, 1), lambda i: (0, 0)),
        ],
        out_shape=[
            jax.ShapeDtypeStruct((_N, 1), jnp.int32),
            jax.ShapeDtypeStruct((_K, _D), jnp.float32),
            jax.ShapeDtypeStruct((1, 1), jnp.float32),
        ],
        scratch_shapes=[
            pltpu.VMEM((_D, _K), jnp.bfloat16),
            pltpu.VMEM((1, _K), jnp.float32),
        ],
    )(x, W)


def _sc_gather(wn, idx):
    info = plsc.get_sparse_core_info()
    nw = info.num_cores * info.num_subcores
    bpw = _N // nw
    nchunk = bpw // _IDX_CHUNK
    mesh = plsc.VectorSubcoreMesh(core_axis_name="c", subcore_axis_name="s")

    @functools.partial(
        pl.kernel,
        mesh=mesh,
        out_type=jax.ShapeDtypeStruct((_N, _D), jnp.float32),
        scratch_types=[
            pltpu.VMEM((nchunk, _IDX_CHUNK), jnp.int32),
            pltpu.VMEM((nchunk, _IDX_CHUNK, _D), jnp.float32),
            pltpu.SemaphoreType.DMA,
        ],
        compiler_params=pltpu.CompilerParams(use_tc_tiling_on_sc=False),
    )
    def gather_kernel(wn_hbm, idx_hbm, out_hbm, idx_v, rows_v, sem):
        wid = lax.axis_index("s") * info.num_cores + lax.axis_index("c")
        base = wid * bpw
        for j in range(nchunk):
            pltpu.sync_copy(
                idx_hbm.at[pl.ds(base + j * _IDX_CHUNK, _IDX_CHUNK)],
                idx_v.at[j])
            pltpu.async_copy(wn_hbm.at[idx_v.at[j]], rows_v.at[j], sem).wait()
            pltpu.sync_copy(
                rows_v.at[j],
                out_hbm.at[pl.ds(base + j * _IDX_CHUNK, _IDX_CHUNK)])

    return gather_kernel(wn, idx)


def kernel(x, W):
    idx2d, wn, loss_sum = _tc_call(x, W)
    idx = idx2d[:, 0]
    quantized = _sc_gather(wn, idx)
    m = loss_sum[0, 0] / jnp.float32(_N * _D)
    loss = m + jnp.float32(_COMMIT) * m
    return (quantized, loss, idx)
